# Initial kernel scaffold; baseline (speedup 1.0000x reference)
import jax
import jax.numpy as jnp
from jax.experimental import pallas as pl


def kernel(x, t_years, edge_index, L_basis, alpha_w, W1, b1, W2, b2):
    raise NotImplementedError("write your pallas kernel here")


# speedup vs baseline: 48.0276x; 48.0276x over previous
import functools
import math

import jax
import jax.numpy as jnp
from jax import lax
from jax.experimental import pallas as pl
from jax.experimental.pallas import tpu as pltpu
from jax.experimental.pallas import tpu_sc as plsc

NV = 4096
E = 65536
BATCH = 8
KB = 5
HID = 16
NT = 16
NC = 2
BH = BATCH // NC
EPT = E // NT
NR = NV // NT
NCHUNK = EPT // 128

_f32 = jnp.float32
_i32 = jnp.int32


def _sc_body(x_hbm, src_hbm, dst_hbm, fc_hbm, out_hbm,
             x4, src1, dst2, fc, dinv_loc, dbuf, msg, gloc,
             sbuf, zfin, s_sh, z_sh, g_sh, dinv_sh):
    c = lax.axis_index("c")
    t = lax.axis_index("s")
    r0 = t * NR
    e0 = t * EPT

    iot = lax.iota(_i32, 16)
    rpat = iot >> 2
    cpat = iot & 3
    zeros16 = jnp.zeros((16,), _f32)
    ones16 = jnp.ones((16,), _f32)

    pltpu.sync_copy(x_hbm.at[c], x4)
    pltpu.sync_copy(src_hbm.at[pl.ds(e0, EPT)], src1)
    pltpu.sync_copy(dst_hbm.at[t], dst2)
    pltpu.sync_copy(fc_hbm, fc)

    def _zero_sbuf(i, _):
        plsc.store_scatter(sbuf, [rpat + i * 4, cpat], zeros16)
        return _
    lax.fori_loop(0, NR * 4 // 16, _zero_sbuf, None)
    pltpu.sync_copy(sbuf, s_sh.at[pl.ds(r0, NR)])
    pltpu.sync_copy(sbuf, z_sh.at[pl.ds(r0, NR)])
    pltpu.sync_copy(sbuf, g_sh.at[pl.ds(r0, NR)])

    def _fill_ones(i, _):
        plsc.store_scatter(msg, [rpat + i * 4, cpat], ones16)
        return _
    lax.fori_loop(0, EPT * 4 // 16, _fill_ones, None)
    plsc.subcore_barrier()

    for j in range(NCHUNK):
        pltpu.sync_copy(msg.at[pl.ds(j * 128, 128)], g_sh.at[dst2.at[j]],
                        add=True)
    plsc.subcore_barrier()

    pltpu.sync_copy(g_sh.at[pl.ds(r0, NR)], sbuf)

    def _dinv_step(i, _):
        d = plsc.load_gather(sbuf, [i * 16 + iot, jnp.zeros((16,), _i32)])
        d = d + 1.0
        yi = 0x5F3759DF - (plsc.bitcast(d, _i32) >> 1)
        y = plsc.bitcast(yi, _f32)
        for _it in range(4):
            y = y * (1.5 - 0.5 * d * y * y)
        dbuf[pl.ds(i * 16, 16)] = y
        return _
    lax.fori_loop(0, NR // 16, _dinv_step, None)
    pltpu.sync_copy(dbuf, dinv_sh.at[pl.ds(r0, NR)])
    plsc.subcore_barrier()
    pltpu.sync_copy(dinv_sh, dinv_loc)

    def _compute_msgs(val2d):
        def body(i, _):
            eidx = i * 4 + rpat
            srcv = plsc.load_gather(src1, [eidx])
            dstv = plsc.load_gather(dst2, [eidx >> 7, eidx & 127])
            nrm = (plsc.load_gather(dinv_loc, [srcv]) *
                   plsc.load_gather(dinv_loc, [dstv]))
            vals = plsc.load_gather(val2d, [srcv, cpat])
            plsc.store_scatter(msg, [eidx, cpat], vals * nrm)
            return _
        lax.fori_loop(0, EPT // 4, body, None)

    _compute_msgs(x4)
    for j in range(NCHUNK):
        pltpu.sync_copy(msg.at[pl.ds(j * 128, 128)], s_sh.at[dst2.at[j]],
                        add=True)
    plsc.subcore_barrier()

    pltpu.sync_copy(s_sh.at[pl.ds(r0, NR)], sbuf)
    a2s = [plsc.load_gather(fc, [jnp.zeros((16,), _i32) + 0,
                                 jnp.zeros((16,), _i32) + j])
           for j in range(HID)]
    b2s = [plsc.load_gather(fc, [jnp.zeros((16,), _i32) + 1,
                                 jnp.zeros((16,), _i32) + j])
           for j in range(HID)]
    cs = [plsc.load_gather(fc, [jnp.zeros((16,), _i32) + 2,
                                jnp.zeros((16,), _i32) + j])
          for j in range(HID)]
    s0 = plsc.load_gather(fc, [jnp.zeros((16,), _i32) + 3,
                               jnp.zeros((16,), _i32)])
    bias2 = plsc.load_gather(fc, [jnp.zeros((16,), _i32) + 3,
                                  jnp.zeros((16,), _i32) + 1])

    def _f_step(i, _):
        rr = rpat + i * 4
        sv = plsc.load_gather(sbuf, [rr, cpat])
        xv = plsc.load_gather(x4, [r0 + rr, cpat])
        di = plsc.load_gather(dbuf, [rr])
        s_full = sv + di * di * xv
        acc = jnp.zeros((16,), _f32)
        for j in range(HID):
            ej = jnp.exp(s_full * a2s[j] + b2s[j])
            acc = acc + cs[j] / (ej + 1.0)
        plsc.store_scatter(zfin, [rr, cpat], s0 - 2.0 * acc)
        return _
    lax.fori_loop(0, NR * 4 // 16, _f_step, None)
    pltpu.sync_copy(zfin, g_sh.at[pl.ds(r0, NR)])
    plsc.subcore_barrier()
    pltpu.sync_copy(g_sh, gloc)

    _compute_msgs(gloc)
    for j in range(NCHUNK):
        pltpu.sync_copy(msg.at[pl.ds(j * 128, 128)], z_sh.at[dst2.at[j]],
                        add=True)
    plsc.subcore_barrier()

    pltpu.sync_copy(z_sh.at[pl.ds(r0, NR)], sbuf)

    def _fin_step(i, _):
        rr = rpat + i * 4
        zv = plsc.load_gather(sbuf, [rr, cpat])
        gv = plsc.load_gather(gloc, [r0 + rr, cpat])
        di = plsc.load_gather(dbuf, [rr])
        plsc.store_scatter(zfin, [rr, cpat], zv + di * di * gv + bias2)
        return _
    lax.fori_loop(0, NR * 4 // 16, _fin_step, None)
    pltpu.sync_copy(zfin, out_hbm.at[c, pl.ds(r0, NR)])


def _graph_sc(x_sc, src, dst3, fcoef):
    mesh = plsc.VectorSubcoreMesh(core_axis_name="c", subcore_axis_name="s")
    f = pl.kernel(
        _sc_body,
        out_type=jax.ShapeDtypeStruct((NC, NV, BH), _f32),
        mesh=mesh,
        compiler_params=pltpu.CompilerParams(
            needs_layout_passes=False, use_tc_tiling_on_sc=False),
        scratch_types=[
            pltpu.VMEM((NV, BH), _f32),
            pltpu.VMEM((EPT,), _i32),
            pltpu.VMEM((NCHUNK, 128), _i32),
            pltpu.VMEM((4, 16), _f32),
            pltpu.VMEM((NV,), _f32),
            pltpu.VMEM((NR,), _f32),
            pltpu.VMEM((EPT, BH), _f32),
            pltpu.VMEM((NV, BH), _f32),
            pltpu.VMEM((NR, BH), _f32),
            pltpu.VMEM((NR, BH), _f32),
            pltpu.VMEM_SHARED((NV, BH), _f32),
            pltpu.VMEM_SHARED((NV, BH), _f32),
            pltpu.VMEM_SHARED((NV, BH), _f32),
            pltpu.VMEM_SHARED((NV,), _f32),
        ],
    )
    return f(x_sc, src, dst3, fcoef)


TILE_U = 256
OMEGA = 2.0 * math.pi


def _tc_body(t_ref, aw_ref, L_ref, xT_ref, g_ref, out_ref):
    k = pl.program_id(1)
    tv = t_ref[...]
    freq = ((k + 1) // 2).astype(_f32)
    ang = OMEGA * freq * tv
    embk = jnp.where(k == 0, jnp.ones_like(tv),
                     jnp.where(k % 2 == 1, jnp.cos(ang), jnp.sin(ang)))
    mm = jnp.dot(L_ref[0], xT_ref[...], preferred_element_type=_f32)
    contrib = mm * embk

    @pl.when(k == 0)
    def _():
        out_ref[...] = contrib

    @pl.when(k > 0)
    def _():
        out_ref[...] += contrib

    @pl.when(k == KB - 1)
    def _():
        acc = jnp.zeros_like(tv) + aw_ref[0]
        acc += aw_ref[1] * jnp.cos(OMEGA * tv)
        acc += aw_ref[2] * jnp.sin(OMEGA * tv)
        acc += aw_ref[3] * jnp.cos(2.0 * OMEGA * tv)
        acc += aw_ref[4] * jnp.sin(2.0 * OMEGA * tv)
        alpha = jax.nn.sigmoid(acc)
        out_ref[...] += alpha * g_ref[...]


def _dxdt_combine(t_years, alpha_w, L_basis, xT, gT):
    grid = (NV // TILE_U, KB)
    return pl.pallas_call(
        _tc_body,
        grid=grid,
        in_specs=[
            pl.BlockSpec((1, BATCH), lambda u, k: (0, 0)),
            pl.BlockSpec(memory_space=pltpu.SMEM),
            pl.BlockSpec((1, TILE_U, NV), lambda u, k: (k, u, 0)),
            pl.BlockSpec((NV, BATCH), lambda u, k: (0, 0)),
            pl.BlockSpec((TILE_U, BATCH), lambda u, k: (u, 0)),
        ],
        out_specs=pl.BlockSpec((TILE_U, BATCH), lambda u, k: (u, 0)),
        out_shape=jax.ShapeDtypeStruct((NV, BATCH), _f32),
    )(t_years.reshape(1, BATCH), alpha_w, L_basis, xT, gT)


@jax.jit
def kernel(x, t_years, edge_index, L_basis, alpha_w, W1, b1, W2, b2):
    src = edge_index[0]
    dst = edge_index[1]
    xT = x.T
    x_sc = jnp.stack([xT[:, :BH], xT[:, BH:]])
    dst3 = dst.reshape(NT, NCHUNK, 128)

    cvec = W2[:, 0]
    row3 = jnp.zeros((16,), _f32).at[0].set(jnp.sum(cvec)).at[1].set(b2[0])
    fcoef = jnp.stack([2.0 * W1[0], 2.0 * b1, cvec, row3])

    g_parts = _graph_sc(x_sc, src, dst3, fcoef)
    gT = jnp.concatenate([g_parts[0], g_parts[1]], axis=1)

    out_T = _dxdt_combine(t_years, alpha_w, L_basis, xT, gT)
    return out_T.T


# speedup vs baseline: 72.7389x; 1.5145x over previous
import functools
import math

import jax
import jax.numpy as jnp
from jax import lax
from jax.experimental import pallas as pl
from jax.experimental.pallas import tpu as pltpu
from jax.experimental.pallas import tpu_sc as plsc

NV = 4096
E = 65536
BATCH = 8
KB = 5
HID = 16
NT = 16
NC = 2
BH = BATCH // NC
EPT = E // NT
NR = NV // NT
NCHUNK = EPT // 128

_f32 = jnp.float32
_i32 = jnp.int32


def _sc_body(x_hbm, src_hbm, dst_hbm, fc_hbm, out_hbm,
             x4, src1, dst2, fc, dinv_loc, dbuf, msg, gloc,
             sbuf, zfin, s_sh, z_sh, g_sh, dinv_sh):
    c = lax.axis_index("c")
    t = lax.axis_index("s")
    r0 = t * NR
    e0 = t * EPT

    iot = lax.iota(_i32, 16)
    rpat = iot >> 2
    cpat = iot & 3
    zeros16 = jnp.zeros((16,), _f32)
    ones16 = jnp.ones((16,), _f32)

    pltpu.sync_copy(x_hbm.at[c], x4)
    pltpu.sync_copy(src_hbm.at[pl.ds(e0, EPT)], src1)
    pltpu.sync_copy(dst_hbm.at[t], dst2)
    pltpu.sync_copy(fc_hbm, fc)

    def _zero_sbuf(i, _):
        plsc.store_scatter(sbuf, [rpat + i * 4, cpat], zeros16)
        return _
    lax.fori_loop(0, NR * 4 // 16, _zero_sbuf, None)
    pltpu.sync_copy(sbuf, s_sh.at[pl.ds(r0, NR)])
    pltpu.sync_copy(sbuf, z_sh.at[pl.ds(r0, NR)])
    pltpu.sync_copy(sbuf, g_sh.at[pl.ds(r0, NR)])

    def _fill_ones(i, _):
        plsc.store_scatter(msg, [rpat + i * 4, cpat], ones16)
        return _
    lax.fori_loop(0, EPT * 4 // 16, _fill_ones, None)
    plsc.subcore_barrier()

    for j in range(NCHUNK):
        pltpu.sync_copy(msg.at[pl.ds(j * 128, 128)], g_sh.at[dst2.at[j]],
                        add=True)
    plsc.subcore_barrier()

    pltpu.sync_copy(g_sh.at[pl.ds(r0, NR)], sbuf)

    def _dinv_step(i, _):
        d = plsc.load_gather(sbuf, [i * 16 + iot, jnp.zeros((16,), _i32)])
        d = d + 1.0
        yi = 0x5F3759DF - (plsc.bitcast(d, _i32) >> 1)
        y = plsc.bitcast(yi, _f32)
        for _it in range(4):
            y = y * (1.5 - 0.5 * d * y * y)
        dbuf[pl.ds(i * 16, 16)] = y
        return _
    lax.fori_loop(0, NR // 16, _dinv_step, None)
    pltpu.sync_copy(dbuf, dinv_sh.at[pl.ds(r0, NR)])
    plsc.subcore_barrier()
    pltpu.sync_copy(dinv_sh, dinv_loc)

    def _compute_msgs(val2d):
        def body(i, _):
            eidx = i * 4 + rpat
            srcv = plsc.load_gather(src1, [eidx])
            dstv = plsc.load_gather(dst2, [eidx >> 7, eidx & 127])
            nrm = (plsc.load_gather(dinv_loc, [srcv]) *
                   plsc.load_gather(dinv_loc, [dstv]))
            vals = plsc.load_gather(val2d, [srcv, cpat])
            plsc.store_scatter(msg, [eidx, cpat], vals * nrm)
            return _
        lax.fori_loop(0, EPT // 4, body, None)

    _compute_msgs(x4)
    for j in range(NCHUNK):
        pltpu.sync_copy(msg.at[pl.ds(j * 128, 128)], s_sh.at[dst2.at[j]],
                        add=True)
    plsc.subcore_barrier()

    pltpu.sync_copy(s_sh.at[pl.ds(r0, NR)], sbuf)
    a2s = [plsc.load_gather(fc, [jnp.zeros((16,), _i32) + 0,
                                 jnp.zeros((16,), _i32) + j])
           for j in range(HID)]
    b2s = [plsc.load_gather(fc, [jnp.zeros((16,), _i32) + 1,
                                 jnp.zeros((16,), _i32) + j])
           for j in range(HID)]
    cs = [plsc.load_gather(fc, [jnp.zeros((16,), _i32) + 2,
                                jnp.zeros((16,), _i32) + j])
          for j in range(HID)]
    s0 = plsc.load_gather(fc, [jnp.zeros((16,), _i32) + 3,
                               jnp.zeros((16,), _i32)])
    bias2 = plsc.load_gather(fc, [jnp.zeros((16,), _i32) + 3,
                                  jnp.zeros((16,), _i32) + 1])

    def _f_step(i, _):
        rr = rpat + i * 4
        sv = plsc.load_gather(sbuf, [rr, cpat])
        xv = plsc.load_gather(x4, [r0 + rr, cpat])
        di = plsc.load_gather(dbuf, [rr])
        s_full = sv + di * di * xv
        acc = jnp.zeros((16,), _f32)
        for j in range(HID):
            ej = jnp.exp(s_full * a2s[j] + b2s[j])
            acc = acc + cs[j] / (ej + 1.0)
        plsc.store_scatter(zfin, [rr, cpat], s0 - 2.0 * acc)
        return _
    lax.fori_loop(0, NR * 4 // 16, _f_step, None)
    pltpu.sync_copy(zfin, g_sh.at[pl.ds(r0, NR)])
    plsc.subcore_barrier()
    pltpu.sync_copy(g_sh, gloc)

    _compute_msgs(gloc)
    for j in range(NCHUNK):
        pltpu.sync_copy(msg.at[pl.ds(j * 128, 128)], z_sh.at[dst2.at[j]],
                        add=True)
    plsc.subcore_barrier()

    pltpu.sync_copy(z_sh.at[pl.ds(r0, NR)], sbuf)

    def _fin_step(i, _):
        rr = rpat + i * 4
        zv = plsc.load_gather(sbuf, [rr, cpat])
        gv = plsc.load_gather(gloc, [r0 + rr, cpat])
        di = plsc.load_gather(dbuf, [rr])
        plsc.store_scatter(zfin, [rr, cpat], zv + di * di * gv + bias2)
        return _
    lax.fori_loop(0, NR * 4 // 16, _fin_step, None)
    pltpu.sync_copy(zfin, out_hbm.at[c, pl.ds(r0, NR)])


def _graph_sc(x_sc, src, dst3, fcoef):
    mesh = plsc.VectorSubcoreMesh(core_axis_name="c", subcore_axis_name="s")
    f = pl.kernel(
        _sc_body,
        out_type=jax.ShapeDtypeStruct((NC, NV, BH), _f32),
        mesh=mesh,
        compiler_params=pltpu.CompilerParams(
            needs_layout_passes=False, use_tc_tiling_on_sc=False),
        scratch_types=[
            pltpu.VMEM((NV, BH), _f32),
            pltpu.VMEM((EPT,), _i32),
            pltpu.VMEM((NCHUNK, 128), _i32),
            pltpu.VMEM((4, 16), _f32),
            pltpu.VMEM((NV,), _f32),
            pltpu.VMEM((NR,), _f32),
            pltpu.VMEM((EPT, BH), _f32),
            pltpu.VMEM((NV, BH), _f32),
            pltpu.VMEM((NR, BH), _f32),
            pltpu.VMEM((NR, BH), _f32),
            pltpu.VMEM_SHARED((NV, BH), _f32),
            pltpu.VMEM_SHARED((NV, BH), _f32),
            pltpu.VMEM_SHARED((NV, BH), _f32),
            pltpu.VMEM_SHARED((NV,), _f32),
        ],
    )
    return f(x_sc, src, dst3, fcoef)


TILE_U = 512
TILE_C = 1024
OMEGA = 2.0 * math.pi


def _dxdt_body(t_ref, L_ref, xT_ref, out_ref):
    k = pl.program_id(1)
    tv = t_ref[...]
    freq = ((k + 1) // 2).astype(_f32)
    ang = OMEGA * freq * tv
    embk = jnp.where(k == 0, jnp.ones_like(tv),
                     jnp.where(k % 2 == 1, jnp.cos(ang), jnp.sin(ang)))
    mm = jnp.dot(L_ref[0], xT_ref[...], preferred_element_type=_f32)
    contrib = mm * embk

    @pl.when(k == 0)
    def _():
        out_ref[...] = contrib

    @pl.when(k > 0)
    def _():
        out_ref[...] += contrib


def _dxdt(t_years, L_basis, xT):
    grid = (NV // TILE_U, KB)
    return pl.pallas_call(
        _dxdt_body,
        grid=grid,
        in_specs=[
            pl.BlockSpec((1, BATCH), lambda u, k: (0, 0)),
            pl.BlockSpec((1, TILE_U, NV), lambda u, k: (k, u, 0)),
            pl.BlockSpec((NV, BATCH), lambda u, k: (0, 0)),
        ],
        out_specs=pl.BlockSpec((TILE_U, BATCH), lambda u, k: (u, 0)),
        out_shape=jax.ShapeDtypeStruct((NV, BATCH), _f32),
    )(t_years.reshape(1, BATCH), L_basis, xT)


def _combine_body(t_ref, aw_ref, dx_ref, g_ref, out_ref):
    tv = t_ref[...]
    acc = jnp.zeros_like(tv) + aw_ref[0]
    acc += aw_ref[1] * jnp.cos(OMEGA * tv)
    acc += aw_ref[2] * jnp.sin(OMEGA * tv)
    acc += aw_ref[3] * jnp.cos(2.0 * OMEGA * tv)
    acc += aw_ref[4] * jnp.sin(2.0 * OMEGA * tv)
    alpha = jax.nn.sigmoid(acc)
    g = jnp.concatenate([g_ref[0], g_ref[1]], axis=1)
    out_ref[...] = dx_ref[...] + alpha * g


def _combine(t_years, alpha_w, dxT, g_parts):
    grid = (NV // TILE_C,)
    return pl.pallas_call(
        _combine_body,
        grid=grid,
        in_specs=[
            pl.BlockSpec((1, BATCH), lambda u: (0, 0)),
            pl.BlockSpec(memory_space=pltpu.SMEM),
            pl.BlockSpec((TILE_C, BATCH), lambda u: (u, 0)),
            pl.BlockSpec((NC, TILE_C, BH), lambda u: (0, u, 0)),
        ],
        out_specs=pl.BlockSpec((TILE_C, BATCH), lambda u: (u, 0)),
        out_shape=jax.ShapeDtypeStruct((NV, BATCH), _f32),
    )(t_years.reshape(1, BATCH), alpha_w, dxT, g_parts)


@jax.jit
def kernel(x, t_years, edge_index, L_basis, alpha_w, W1, b1, W2, b2):
    src = edge_index[0]
    dst = edge_index[1]
    xT = x.T
    x_sc = jnp.stack([xT[:, :BH], xT[:, BH:]])
    dst3 = dst.reshape(NT, NCHUNK, 128)

    cvec = W2[:, 0]
    row3 = jnp.zeros((16,), _f32).at[0].set(jnp.sum(cvec)).at[1].set(b2[0])
    fcoef = jnp.stack([2.0 * W1[0], 2.0 * b1, cvec, row3])

    g_parts = _graph_sc(x_sc, src, dst3, fcoef)
    dxT = _dxdt(t_years, L_basis, xT)
    out_T = _combine(t_years, alpha_w, dxT, g_parts)
    return out_T.T


# speedup vs baseline: 72.9922x; 1.0035x over previous
import functools
import math

import jax
import jax.numpy as jnp
from jax import lax
from jax.experimental import pallas as pl
from jax.experimental.pallas import tpu as pltpu
from jax.experimental.pallas import tpu_sc as plsc

NV = 4096
E = 65536
BATCH = 8
KB = 5
HID = 16
NT = 16
NC = 2
BH = BATCH // NC
EPT = E // NT
NR = NV // NT
NCHUNK = EPT // 128

_f32 = jnp.float32
_i32 = jnp.int32


def _sc_body(x_hbm, ei_hbm, dst_hbm, fc_hbm, out_hbm,
             x4, src1, dst2, fc, dinv_loc, dbuf, msg,
             gloc, sbuf, zfin, s_sh, z_sh, g_sh, dinv_sh):
    c = lax.axis_index("c")
    t = lax.axis_index("s")
    r0 = t * NR
    e0 = t * EPT

    iot = lax.iota(_i32, 16)
    rpat = iot >> 2
    cpat = iot & 3
    zeros16i = jnp.zeros((16,), _i32)
    zeros16 = jnp.zeros((16,), _f32)
    ones16 = jnp.ones((16,), _f32)

    pltpu.sync_copy(x_hbm.at[pl.ds(c * BH, BH)], x4)
    pltpu.sync_copy(ei_hbm.at[0, pl.ds(e0, EPT)], src1)
    pltpu.sync_copy(dst_hbm.at[t], dst2)
    pltpu.sync_copy(fc_hbm, fc)

    def _zero_sbuf(i, _):
        plsc.store_scatter(sbuf, [rpat + i * 4, cpat], zeros16)
        return _
    lax.fori_loop(0, NR * 4 // 16, _zero_sbuf, None)
    pltpu.sync_copy(sbuf, s_sh.at[pl.ds(r0, NR)])
    pltpu.sync_copy(sbuf, z_sh.at[pl.ds(r0, NR)])
    pltpu.sync_copy(sbuf, g_sh.at[pl.ds(r0, NR)])

    def _fill_ones(i, _):
        plsc.store_scatter(msg, [rpat + i * 4, cpat], ones16)
        return _
    lax.fori_loop(0, EPT * 4 // 16, _fill_ones, None)
    plsc.subcore_barrier()

    for j in range(NCHUNK):
        pltpu.sync_copy(msg.at[pl.ds(j * 128, 128)], g_sh.at[dst2.at[j]],
                        add=True)
    plsc.subcore_barrier()

    pltpu.sync_copy(g_sh.at[pl.ds(r0, NR)], sbuf)

    def _dinv_step(i, _):
        d = plsc.load_gather(sbuf, [i * 16 + iot, zeros16i])
        d = d + 1.0
        yi = 0x5F3759DF - (plsc.bitcast(d, _i32) >> 1)
        y = plsc.bitcast(yi, _f32)
        for _it in range(4):
            y = y * (1.5 - 0.5 * d * y * y)
        dbuf[pl.ds(i * 16, 16)] = y
        return _
    lax.fori_loop(0, NR // 16, _dinv_step, None)
    pltpu.sync_copy(dbuf, dinv_sh.at[pl.ds(r0, NR)])
    plsc.subcore_barrier()
    pltpu.sync_copy(dinv_sh, dinv_loc)

    def _compute_msgs(val_gather):
        def body(i, _):
            eidx = i * 4 + rpat
            srcv = plsc.load_gather(src1, [eidx])
            dstv = plsc.load_gather(dst2, [eidx >> 7, eidx & 127])
            nrm = (plsc.load_gather(dinv_loc, [srcv]) *
                   plsc.load_gather(dinv_loc, [dstv]))
            vals = val_gather(srcv)
            plsc.store_scatter(msg, [eidx, cpat], vals * nrm)
            return _
        lax.fori_loop(0, EPT // 4, body, None)

    _compute_msgs(lambda srcv: plsc.load_gather(x4, [cpat, srcv]))
    for j in range(NCHUNK):
        pltpu.sync_copy(msg.at[pl.ds(j * 128, 128)], s_sh.at[dst2.at[j]],
                        add=True)
    plsc.subcore_barrier()

    pltpu.sync_copy(s_sh.at[pl.ds(r0, NR)], sbuf)
    a2s = [plsc.load_gather(fc, [zeros16i, zeros16i + j]) for j in range(HID)]
    b2s = [plsc.load_gather(fc, [zeros16i + 1, zeros16i + j])
           for j in range(HID)]
    cs = [plsc.load_gather(fc, [zeros16i + 2, zeros16i + j])
          for j in range(HID)]
    s0 = cs[0]
    for j in range(1, HID):
        s0 = s0 + cs[j]
    bias2 = plsc.load_gather(fc, [zeros16i + 3, zeros16i])
    alph = plsc.load_gather(fc, [zeros16i + 3, 1 + c * BH + cpat])

    def _f_step(i, _):
        rr = rpat + i * 4
        sv = plsc.load_gather(sbuf, [rr, cpat])
        xv = plsc.load_gather(x4, [cpat, r0 + rr])
        di = plsc.load_gather(dbuf, [rr])
        s_full = sv + di * di * xv
        acc = jnp.zeros((16,), _f32)
        for j in range(HID):
            ej = jnp.exp(s_full * a2s[j] + b2s[j])
            acc = acc + cs[j] / (ej + 1.0)
        plsc.store_scatter(zfin, [rr, cpat], s0 - 2.0 * acc)
        return _
    lax.fori_loop(0, NR * 4 // 16, _f_step, None)
    pltpu.sync_copy(zfin, g_sh.at[pl.ds(r0, NR)])
    plsc.subcore_barrier()
    pltpu.sync_copy(g_sh, gloc)

    _compute_msgs(lambda srcv: plsc.load_gather(gloc, [srcv, cpat]))
    for j in range(NCHUNK):
        pltpu.sync_copy(msg.at[pl.ds(j * 128, 128)], z_sh.at[dst2.at[j]],
                        add=True)
    plsc.subcore_barrier()

    pltpu.sync_copy(z_sh.at[pl.ds(r0, NR)], sbuf)

    def _fin_step(i, _):
        rr = rpat + i * 4
        zv = plsc.load_gather(sbuf, [rr, cpat])
        gv = plsc.load_gather(gloc, [r0 + rr, cpat])
        di = plsc.load_gather(dbuf, [rr])
        plsc.store_scatter(zfin, [rr, cpat],
                           alph * (zv + di * di * gv + bias2))
        return _
    lax.fori_loop(0, NR * 4 // 16, _fin_step, None)
    pltpu.sync_copy(zfin, out_hbm.at[c, pl.ds(r0, NR)])


def _graph_sc(x, ei, dst3, fcoef):
    mesh = plsc.VectorSubcoreMesh(core_axis_name="c", subcore_axis_name="s")
    f = pl.kernel(
        _sc_body,
        out_type=jax.ShapeDtypeStruct((NC, NV, BH), _f32),
        mesh=mesh,
        compiler_params=pltpu.CompilerParams(
            needs_layout_passes=False, use_tc_tiling_on_sc=False),
        scratch_types=[
            pltpu.VMEM((BH, NV), _f32),
            pltpu.VMEM((EPT,), _i32),
            pltpu.VMEM((NCHUNK, 128), _i32),
            pltpu.VMEM((4, 16), _f32),
            pltpu.VMEM((NV,), _f32),
            pltpu.VMEM((NR,), _f32),
            pltpu.VMEM((EPT, BH), _f32),
            pltpu.VMEM((NV, BH), _f32),
            pltpu.VMEM((NR, BH), _f32),
            pltpu.VMEM((NR, BH), _f32),
            pltpu.VMEM_SHARED((NV, BH), _f32),
            pltpu.VMEM_SHARED((NV, BH), _f32),
            pltpu.VMEM_SHARED((NV, BH), _f32),
            pltpu.VMEM_SHARED((NV,), _f32),
        ],
    )
    return f(x, ei, dst3, fcoef)


TILE_U = 512
OMEGA = 2.0 * math.pi


def _dxdt_body(t_ref, L_ref, xT_ref, out_ref):
    k = pl.program_id(1)
    tv = t_ref[...]
    freq = ((k + 1) // 2).astype(_f32)
    ang = OMEGA * freq * tv
    embk = jnp.where(k == 0, jnp.ones_like(tv),
                     jnp.where(k % 2 == 1, jnp.cos(ang), jnp.sin(ang)))
    mm = jnp.dot(L_ref[0], xT_ref[...], preferred_element_type=_f32)
    contrib = mm * embk

    @pl.when(k == 0)
    def _():
        out_ref[...] = contrib

    @pl.when(k > 0)
    def _():
        out_ref[...] += contrib


def _dxdt(t_years, L_basis, xT):
    grid = (NV // TILE_U, KB)
    return pl.pallas_call(
        _dxdt_body,
        grid=grid,
        in_specs=[
            pl.BlockSpec((1, BATCH), lambda u, k: (0, 0)),
            pl.BlockSpec((1, TILE_U, NV), lambda u, k: (k, u, 0)),
            pl.BlockSpec((NV, BATCH), lambda u, k: (0, 0)),
        ],
        out_specs=pl.BlockSpec((TILE_U, BATCH), lambda u, k: (u, 0)),
        out_shape=jax.ShapeDtypeStruct((NV, BATCH), _f32),
    )(t_years.reshape(1, BATCH), L_basis, xT)


@jax.jit
def kernel(x, t_years, edge_index, L_basis, alpha_w, W1, b1, W2, b2):
    dst3 = edge_index[1].reshape(NT, NCHUNK, 128)
    xT = x.T

    ang = OMEGA * t_years
    alpha = jax.nn.sigmoid(
        alpha_w[0]
        + alpha_w[1] * jnp.cos(ang) + alpha_w[2] * jnp.sin(ang)
        + alpha_w[3] * jnp.cos(2.0 * ang) + alpha_w[4] * jnp.sin(2.0 * ang))

    row3 = jnp.concatenate([b2, alpha, jnp.zeros((7,), _f32)])
    fcoef = jnp.stack([2.0 * W1[0], 2.0 * b1, W2[:, 0], row3])

    ag = _graph_sc(x, edge_index, dst3, fcoef)
    gT = jnp.concatenate([ag[0], ag[1]], axis=1)
    dxT = _dxdt(t_years, L_basis, xT)
    return (dxT + gT).T
